# baseline (speedup 1.0000x reference)
import jax
import jax.numpy as jnp
from jax.experimental import pallas as pl

VOXEL_SIZE = jnp.array([0.025, 0.025, 6.0], dtype=jnp.float32)
PC_RANGE_MIN = jnp.array([-3.2, -3.2, -3.0], dtype=jnp.float32)
GRID = (256, 256)
FEAT = 32
IN_FEAT = 9


def _relu_kernel(x_ref, o_ref):
    o_ref[...] = jnp.maximum(x_ref[...], 0.0)


def _pallas_relu(x):
    n, c = x.shape
    blk = 10000
    return pl.pallas_call(
        _relu_kernel,
        grid=(n // blk,),
        in_specs=[pl.BlockSpec((blk, c), lambda i: (i, 0))],
        out_specs=pl.BlockSpec((blk, c), lambda i: (i, 0)),
        out_shape=jax.ShapeDtypeStruct(x.shape, x.dtype),
    )(x)


def _pillar_features(pts, W, b):
    HW = GRID[0] * GRID[1]
    coords = jnp.floor((pts - PC_RANGE_MIN) / VOXEL_SIZE).astype(jnp.int32)
    cx = jnp.clip(coords[:, 0], 0, GRID[0] - 1)
    cy = jnp.clip(coords[:, 1], 0, GRID[1] - 1)
    flat = cx * GRID[1] + cy
    ones = jnp.ones((pts.shape[0],), dtype=jnp.float32)
    counts = jax.ops.segment_sum(ones, flat, num_segments=HW)
    sums = jax.ops.segment_sum(pts, flat, num_segments=HW)
    means = sums / jnp.maximum(counts, 1.0)[:, None]
    f_cluster = pts - means[flat]
    centers_xy = (jnp.stack([cx, cy], axis=1).astype(jnp.float32) + 0.5) * VOXEL_SIZE[:2] + PC_RANGE_MIN[:2]
    f_center_xy = pts[:, :2] - centers_xy
    f_center_z = pts[:, 2:3] - (PC_RANGE_MIN[2] + 0.5 * VOXEL_SIZE[2])
    feats_in = jnp.concatenate([pts, f_cluster, f_center_xy, f_center_z], axis=1)
    point_feats = _pallas_relu(feats_in @ W + b)
    voxel_feats = jax.ops.segment_max(point_feats, flat, num_segments=HW)
    voxel_feats = jnp.where(counts[:, None] > 0, voxel_feats, 0.0)
    return voxel_feats, point_feats


def kernel(pc0s, pc1s, W, b):
    frames = [pc0s, pc1s]
    grids = []
    pc0_point_feats = []
    for t, pc in enumerate(frames):
        per_batch = []
        for bi in range(pc.shape[0]):
            vf, pf = _pillar_features(pc[bi], W, b)
            per_batch.append(vf)
            if t == 0:
                pc0_point_feats.append(pf)
        grids.append(jnp.stack(per_batch, axis=0))
    dense_4d = jnp.stack(grids, axis=0)
    pc0_feats = jnp.stack(pc0_point_feats, axis=0)
    return dense_4d, pc0_feats


# speedup vs baseline: 1.7776x; 1.7776x over previous
import functools

import jax
import jax.numpy as jnp
from jax import lax
from jax.experimental import pallas as pl
from jax.experimental.pallas import tpu as pltpu
from jax.experimental.pallas import tpu_sc as plsc

N = 100000
NP = 100096
HW = 256 * 256
C = 32
L = 16
TV = 391
TP = TV * L
NV = N // L
ROWS = HW // 16

_mesh = plsc.VectorSubcoreMesh(core_axis_name="c", subcore_axis_name="s",
                               num_cores=2, num_subcores=16)
_sc_params = pltpu.CompilerParams(needs_layout_passes=False,
                                  use_tc_tiling_on_sc=False)


def _flat_from_xy(x, y):
    cx = ((x + 3.2) / 0.025).astype(jnp.int32)
    cy = ((y + 3.2) / 0.025).astype(jnp.int32)
    cx = jnp.clip(cx, 0, 255)
    cy = jnp.clip(cy, 0, 255)
    return cx * 256 + cy


@functools.partial(
    pl.kernel,
    mesh=_mesh,
    compiler_params=_sc_params,
    out_type=[
        jax.ShapeDtypeStruct((4, NP), jnp.int32),
        jax.ShapeDtypeStruct((4, 5, NP), jnp.float32),
    ],
    scratch_types=[
        pltpu.VMEM((HW // 2,), jnp.float32),
        pltpu.VMEM((TP,), jnp.float32),
        pltpu.VMEM((TP,), jnp.float32),
        pltpu.VMEM((TP,), jnp.float32),
        pltpu.VMEM((TP,), jnp.int32),
        pltpu.VMEM((TP,), jnp.float32),
        pltpu.VMEM((ROWS,), jnp.float32),
        pltpu.VMEM((ROWS,), jnp.float32),
        pltpu.VMEM((ROWS,), jnp.float32),
        pltpu.VMEM((ROWS,), jnp.float32),
        pltpu.VMEM((ROWS,), jnp.float32),
        pltpu.VMEM_SHARED((8, HW // 2), jnp.float32),
    ],
)
def _stats_kernel(pts, flat_out, f5_out,
                  acc, xb, yb, zb, flb, ob, stage, cnt, sx, sy, sz, S):
    cid = lax.axis_index("c")
    sid = lax.axis_index("s")
    base = pl.multiple_of(sid * TP, L)
    valid_bound = jnp.minimum(jnp.maximum(N - sid * TP, 0), TP)
    HH = HW // 2
    myhalf = sid // 8
    lrow = pl.multiple_of((sid % 8) * ROWS, L)

    def cloud_body(i, _):
        cloud = cid * 2 + i
        pltpu.sync_copy(pts.at[cloud, 0, pl.ds(base, TP)], xb)
        pltpu.sync_copy(pts.at[cloud, 1, pl.ds(base, TP)], yb)
        pltpu.sync_copy(pts.at[cloud, 2, pl.ds(base, TP)], zb)

        def fl_body(j, _):
            o = pl.multiple_of(j * L, L)
            flb[pl.ds(o, L)] = _flat_from_xy(xb[pl.ds(o, L)], yb[pl.ds(o, L)])
            return 0
        lax.fori_loop(0, TV, fl_body, 0)

        lane = jnp.arange(L, dtype=jnp.int32)
        for comp in range(4):
            dst = (cnt, sx, sy, sz)[comp]
            for half in range(2):
                def z_body(j, _):
                    for u in range(8):
                        o = pl.multiple_of(j * (8 * L) + u * L, L)
                        acc[pl.ds(o, L)] = jnp.zeros((L,), jnp.float32)
                    return 0
                lax.fori_loop(0, HH // (8 * L), z_body, 0)

                def sc_body(j, _):
                    o = pl.multiple_of(j * L, L)
                    iv = flb[pl.ds(o, L)]
                    if comp == 0:
                        vv = jnp.ones((L,), jnp.float32)
                    elif comp == 1:
                        vv = xb[pl.ds(o, L)]
                    elif comp == 2:
                        vv = yb[pl.ds(o, L)]
                    else:
                        vv = zb[pl.ds(o, L)]
                    m = ((o + lane) < valid_bound) & ((iv >> 15) == half)
                    plsc.addupdate_scatter(acc, [iv & (HH - 1)], vv, mask=m)
                    return 0
                lax.fori_loop(0, TV, sc_body, 0)

                for rnd in range(2):
                    @pl.when((sid // 8) == rnd)
                    def _():
                        pltpu.sync_copy(acc, S.at[sid % 8])
                    plsc.subcore_barrier()

                    @pl.when(myhalf == half)
                    def _():
                        for j in range(8):
                            pltpu.sync_copy(S.at[j, pl.ds(lrow, ROWS)], stage)

                            def mg_body(t, _):
                                for u in range(4):
                                    o = pl.multiple_of(t * (4 * L) + u * L, L)
                                    if rnd == 0 and j == 0:
                                        dst[pl.ds(o, L)] = stage[pl.ds(o, L)]
                                    else:
                                        dst[pl.ds(o, L)] = (dst[pl.ds(o, L)]
                                                            + stage[pl.ds(o, L)])
                                return 0
                            lax.fori_loop(0, ROWS // (4 * L), mg_body, 0)
                    plsc.subcore_barrier()

        def mean_body(t, _):
            o = pl.multiple_of(t * L, L)
            d = jnp.maximum(cnt[pl.ds(o, L)], 1.0)
            sx[pl.ds(o, L)] = sx[pl.ds(o, L)] / d
            sy[pl.ds(o, L)] = sy[pl.ds(o, L)] / d
            sz[pl.ds(o, L)] = sz[pl.ds(o, L)] / d
            return 0
        lax.fori_loop(0, ROWS // L, mean_body, 0)
        pltpu.sync_copy(sx, S.at[0 * 2 + myhalf, pl.ds(lrow, ROWS)])
        pltpu.sync_copy(sy, S.at[1 * 2 + myhalf, pl.ds(lrow, ROWS)])
        pltpu.sync_copy(sz, S.at[2 * 2 + myhalf, pl.ds(lrow, ROWS)])
        plsc.subcore_barrier()

        for comp in range(3):
            src = (xb, yb, zb)[comp]
            for half in range(2):
                pltpu.sync_copy(S.at[comp * 2 + half], acc)

                def g_body(j, _):
                    o = pl.multiple_of(j * L, L)
                    iv = flb[pl.ds(o, L)]
                    g = plsc.load_gather(acc, [iv & (HH - 1)])
                    r = src[pl.ds(o, L)] - g
                    if half == 0:
                        ob[pl.ds(o, L)] = r
                    else:
                        keep = (iv >> 15) == 1
                        ob[pl.ds(o, L)] = jnp.where(keep, r, ob[pl.ds(o, L)])
                    return 0
                lax.fori_loop(0, TV, g_body, 0)
            pltpu.sync_copy(ob, f5_out.at[cloud, comp, pl.ds(base, TP)])

        def fc_body(j, _):
            o = pl.multiple_of(j * L, L)
            fl = flb[pl.ds(o, L)]
            cxf = (fl >> 8).astype(jnp.float32)
            ob[pl.ds(o, L)] = xb[pl.ds(o, L)] - ((cxf + 0.5) * 0.025 + (-3.2))
            return 0
        lax.fori_loop(0, TV, fc_body, 0)
        pltpu.sync_copy(ob, f5_out.at[cloud, 3, pl.ds(base, TP)])

        def fcy_body(j, _):
            o = pl.multiple_of(j * L, L)
            fl = flb[pl.ds(o, L)]
            cyf = (fl & 255).astype(jnp.float32)
            ob[pl.ds(o, L)] = yb[pl.ds(o, L)] - ((cyf + 0.5) * 0.025 + (-3.2))
            return 0
        lax.fori_loop(0, TV, fcy_body, 0)
        pltpu.sync_copy(ob, f5_out.at[cloud, 4, pl.ds(base, TP)])

        pltpu.sync_copy(flb, flat_out.at[cloud, pl.ds(base, TP)])
        plsc.subcore_barrier()
        return 0

    lax.fori_loop(0, 2, cloud_body, 0)


BLK = 8192


def _mm_kernel(pts_ref, f5_ref, w_ref, b_ref, ptfT_ref, pc0_ref):
    x3 = pts_ref[0]
    f5 = f5_ref[0]
    x8 = jnp.concatenate([x3, f5], axis=0)
    w = w_ref[...]
    out = jnp.broadcast_to(b_ref[...].reshape(C, 1), (C, x8.shape[1]))
    for k in range(8):
        out = out + w[:, k:k + 1] * x8[k:k + 1, :]
    out = jnp.maximum(out, 0.0)
    ptfT_ref[0] = out
    pc0_ref[0] = out.T


def _matmul(pts_soa, f5, W, b):
    W8 = jnp.concatenate(
        [W[0:2], (W[2:3] + W[8:9]), W[3:8]], axis=0)
    W8T = W8.T
    nb = (NP + BLK - 1) // BLK
    return pl.pallas_call(
        _mm_kernel,
        grid=(4, nb),
        in_specs=[
            pl.BlockSpec((1, 3, BLK), lambda c, j: (c, 0, j)),
            pl.BlockSpec((1, 5, BLK), lambda c, j: (c, 0, j)),
            pl.BlockSpec((C, 8), lambda c, j: (0, 0)),
            pl.BlockSpec((1, C), lambda c, j: (0, 0)),
        ],
        out_specs=[
            pl.BlockSpec((1, C, BLK), lambda c, j: (c, 0, j)),
            pl.BlockSpec((1, BLK, C), lambda c, j: (c, j, 0)),
        ],
        out_shape=[
            jax.ShapeDtypeStruct((4, C, NP), jnp.float32),
            jax.ShapeDtypeStruct((4, NP, C), jnp.float32),
        ],
    )(pts_soa, f5, W8T, b.reshape(1, C))


SUBV = 625


@functools.partial(
    pl.kernel,
    mesh=_mesh,
    compiler_params=_sc_params,
    out_type=jax.ShapeDtypeStruct((4, C, HW), jnp.float32),
    scratch_types=[
        pltpu.VMEM((HW,), jnp.float32),
        pltpu.VMEM((SUBV * L,), jnp.int32),
        pltpu.VMEM((SUBV * L,), jnp.float32),
    ],
)
def _smax_kernel(flat, ptfT, dense_out, slab, flb, vb):
    cid = lax.axis_index("c")
    sid = lax.axis_index("s")
    ch = sid * 2 + cid

    def cloud_body(cloud, _):
        def z_body(j, _):
            for u in range(8):
                o = pl.multiple_of(j * (8 * L) + u * L, L)
                slab[pl.ds(o, L)] = jnp.zeros((L,), jnp.float32)
            return 0
        lax.fori_loop(0, HW // (8 * L), z_body, 0)

        def chunk_body(k, _):
            cbase = pl.multiple_of(k * (SUBV * L), L)
            pltpu.sync_copy(flat.at[cloud, pl.ds(cbase, SUBV * L)], flb)
            pltpu.sync_copy(ptfT.at[cloud, ch, pl.ds(cbase, SUBV * L)], vb)

            def v_body(j, _):
                o = pl.multiple_of(j * L, L)
                iv = flb[pl.ds(o, L)]
                vv = vb[pl.ds(o, L)]
                cur = plsc.load_gather(slab, [iv])

                def cond(cur):
                    return jnp.any(cur < vv)

                def body(cur):
                    plsc.store_scatter(slab, [iv], jnp.maximum(cur, vv),
                                       mask=cur < vv)
                    return plsc.load_gather(slab, [iv])

                lax.while_loop(cond, body, cur)
                return 0
            lax.fori_loop(0, SUBV, v_body, 0)
            return 0
        lax.fori_loop(0, NV // SUBV, chunk_body, 0)

        pltpu.sync_copy(slab, dense_out.at[cloud, ch])
        return 0

    lax.fori_loop(0, 4, cloud_body, 0)


def kernel(pc0s, pc1s, W, b):
    pts_all = jnp.concatenate([pc0s, pc1s], axis=0)
    pts_soa = jnp.transpose(pts_all, (0, 2, 1))
    pts_soa = jnp.pad(pts_soa, ((0, 0), (0, 0), (0, NP - N)))
    flat, f5 = _stats_kernel(pts_soa)
    ptfT, pc0f = _matmul(pts_soa, f5, W, b)
    denseT = _smax_kernel(flat, ptfT)
    dense_4d = denseT.reshape(2, 2, C, HW).transpose(0, 1, 3, 2)
    pc0_feats = pc0f[:2, :N, :]
    return dense_4d, pc0_feats


# speedup vs baseline: 3.4751x; 1.9549x over previous
import functools

import jax
import jax.numpy as jnp
from jax import lax
from jax.experimental import pallas as pl
from jax.experimental.pallas import tpu as pltpu
from jax.experimental.pallas import tpu_sc as plsc

N = 100000
NP = 100096
HW = 256 * 256
C = 32
L = 16
TV = 391
TP = TV * L
NV = N // L
ROWS = HW // 16

_mesh = plsc.VectorSubcoreMesh(core_axis_name="c", subcore_axis_name="s",
                               num_cores=2, num_subcores=16)
_sc_params = pltpu.CompilerParams(needs_layout_passes=False,
                                  use_tc_tiling_on_sc=False)


def _flat_from_xy(x, y):
    cx = ((x + 3.2) / 0.025).astype(jnp.int32)
    cy = ((y + 3.2) / 0.025).astype(jnp.int32)
    cx = jnp.clip(cx, 0, 255)
    cy = jnp.clip(cy, 0, 255)
    return cx * 256 + cy


@functools.partial(
    pl.kernel,
    mesh=_mesh,
    compiler_params=_sc_params,
    out_type=[
        jax.ShapeDtypeStruct((4, NP), jnp.int32),
        jax.ShapeDtypeStruct((4, 5, NP), jnp.float32),
    ],
    scratch_types=[
        pltpu.VMEM((HW // 2,), jnp.float32),
        pltpu.VMEM((TP,), jnp.float32),
        pltpu.VMEM((TP,), jnp.float32),
        pltpu.VMEM((TP,), jnp.float32),
        pltpu.VMEM((TP,), jnp.int32),
        pltpu.VMEM((TP,), jnp.float32),
        pltpu.VMEM((ROWS,), jnp.float32),
        pltpu.VMEM((ROWS,), jnp.float32),
        pltpu.VMEM((ROWS,), jnp.float32),
        pltpu.VMEM((ROWS,), jnp.float32),
        pltpu.VMEM((ROWS,), jnp.float32),
        pltpu.VMEM_SHARED((8, HW // 2), jnp.float32),
    ],
)
def _stats_kernel(pts, flat_out, f5_out,
                  acc, xb, yb, zb, flb, ob, stage, cnt, sx, sy, sz, S):
    cid = lax.axis_index("c")
    sid = lax.axis_index("s")
    base = pl.multiple_of(sid * TP, L)
    valid_bound = jnp.minimum(jnp.maximum(N - sid * TP, 0), TP)
    HH = HW // 2
    myhalf = sid // 8
    lrow = pl.multiple_of((sid % 8) * ROWS, L)

    def cloud_body(i, _):
        cloud = cid * 2 + i
        pltpu.sync_copy(pts.at[cloud, 0, pl.ds(base, TP)], xb)
        pltpu.sync_copy(pts.at[cloud, 1, pl.ds(base, TP)], yb)
        pltpu.sync_copy(pts.at[cloud, 2, pl.ds(base, TP)], zb)

        def fl_body(j, _):
            o = pl.multiple_of(j * L, L)
            flb[pl.ds(o, L)] = _flat_from_xy(xb[pl.ds(o, L)], yb[pl.ds(o, L)])
            return 0
        lax.fori_loop(0, TV, fl_body, 0)

        lane = jnp.arange(L, dtype=jnp.int32)
        for comp in range(4):
            dst = (cnt, sx, sy, sz)[comp]
            for half in range(2):
                def z_body(j, _):
                    for u in range(8):
                        o = pl.multiple_of(j * (8 * L) + u * L, L)
                        acc[pl.ds(o, L)] = jnp.zeros((L,), jnp.float32)
                    return 0
                lax.fori_loop(0, HH // (8 * L), z_body, 0)

                def sc_body(j, _):
                    o = pl.multiple_of(j * L, L)
                    iv = flb[pl.ds(o, L)]
                    if comp == 0:
                        vv = jnp.ones((L,), jnp.float32)
                    elif comp == 1:
                        vv = xb[pl.ds(o, L)]
                    elif comp == 2:
                        vv = yb[pl.ds(o, L)]
                    else:
                        vv = zb[pl.ds(o, L)]
                    m = ((o + lane) < valid_bound) & ((iv >> 15) == half)
                    plsc.addupdate_scatter(acc, [iv & (HH - 1)], vv, mask=m)
                    return 0
                lax.fori_loop(0, TV, sc_body, 0)

                for rnd in range(2):
                    @pl.when((sid // 8) == rnd)
                    def _():
                        pltpu.sync_copy(acc, S.at[sid % 8])
                    plsc.subcore_barrier()

                    @pl.when(myhalf == half)
                    def _():
                        for j in range(8):
                            pltpu.sync_copy(S.at[j, pl.ds(lrow, ROWS)], stage)

                            def mg_body(t, _):
                                for u in range(4):
                                    o = pl.multiple_of(t * (4 * L) + u * L, L)
                                    if rnd == 0 and j == 0:
                                        dst[pl.ds(o, L)] = stage[pl.ds(o, L)]
                                    else:
                                        dst[pl.ds(o, L)] = (dst[pl.ds(o, L)]
                                                            + stage[pl.ds(o, L)])
                                return 0
                            lax.fori_loop(0, ROWS // (4 * L), mg_body, 0)
                    plsc.subcore_barrier()

        def mean_body(t, _):
            o = pl.multiple_of(t * L, L)
            d = jnp.maximum(cnt[pl.ds(o, L)], 1.0)
            sx[pl.ds(o, L)] = sx[pl.ds(o, L)] / d
            sy[pl.ds(o, L)] = sy[pl.ds(o, L)] / d
            sz[pl.ds(o, L)] = sz[pl.ds(o, L)] / d
            return 0
        lax.fori_loop(0, ROWS // L, mean_body, 0)
        pltpu.sync_copy(sx, S.at[0 * 2 + myhalf, pl.ds(lrow, ROWS)])
        pltpu.sync_copy(sy, S.at[1 * 2 + myhalf, pl.ds(lrow, ROWS)])
        pltpu.sync_copy(sz, S.at[2 * 2 + myhalf, pl.ds(lrow, ROWS)])
        plsc.subcore_barrier()

        for comp in range(3):
            src = (xb, yb, zb)[comp]
            for half in range(2):
                pltpu.sync_copy(S.at[comp * 2 + half], acc)

                def g_body(j, _):
                    o = pl.multiple_of(j * L, L)
                    iv = flb[pl.ds(o, L)]
                    g = plsc.load_gather(acc, [iv & (HH - 1)])
                    r = src[pl.ds(o, L)] - g
                    if half == 0:
                        ob[pl.ds(o, L)] = r
                    else:
                        keep = (iv >> 15) == 1
                        ob[pl.ds(o, L)] = jnp.where(keep, r, ob[pl.ds(o, L)])
                    return 0
                lax.fori_loop(0, TV, g_body, 0)
            pltpu.sync_copy(ob, f5_out.at[cloud, comp, pl.ds(base, TP)])

        def fc_body(j, _):
            o = pl.multiple_of(j * L, L)
            fl = flb[pl.ds(o, L)]
            cxf = (fl >> 8).astype(jnp.float32)
            ob[pl.ds(o, L)] = xb[pl.ds(o, L)] - ((cxf + 0.5) * 0.025 + (-3.2))
            return 0
        lax.fori_loop(0, TV, fc_body, 0)
        pltpu.sync_copy(ob, f5_out.at[cloud, 3, pl.ds(base, TP)])

        def fcy_body(j, _):
            o = pl.multiple_of(j * L, L)
            fl = flb[pl.ds(o, L)]
            cyf = (fl & 255).astype(jnp.float32)
            ob[pl.ds(o, L)] = yb[pl.ds(o, L)] - ((cyf + 0.5) * 0.025 + (-3.2))
            return 0
        lax.fori_loop(0, TV, fcy_body, 0)
        pltpu.sync_copy(ob, f5_out.at[cloud, 4, pl.ds(base, TP)])

        pltpu.sync_copy(flb, flat_out.at[cloud, pl.ds(base, TP)])
        plsc.subcore_barrier()
        return 0

    lax.fori_loop(0, 2, cloud_body, 0)


BLK = 8192


def _mm_kernel(pts_ref, f5_ref, w_ref, b_ref, ptfT_ref, pc0_ref):
    x3 = pts_ref[0]
    f5 = f5_ref[0]
    x8 = jnp.concatenate([x3, f5], axis=0)
    w = w_ref[...]
    out = jnp.broadcast_to(b_ref[...].reshape(C, 1), (C, x8.shape[1]))
    for k in range(8):
        out = out + w[:, k:k + 1] * x8[k:k + 1, :]
    out = jnp.maximum(out, 0.0)
    ptfT_ref[0] = out
    pc0_ref[0] = out.T


def _matmul(pts_soa, f5, W, b):
    W8 = jnp.concatenate(
        [W[0:2], (W[2:3] + W[8:9]), W[3:8]], axis=0)
    W8T = W8.T
    nb = (NP + BLK - 1) // BLK
    return pl.pallas_call(
        _mm_kernel,
        grid=(4, nb),
        in_specs=[
            pl.BlockSpec((1, 3, BLK), lambda c, j: (c, 0, j)),
            pl.BlockSpec((1, 5, BLK), lambda c, j: (c, 0, j)),
            pl.BlockSpec((C, 8), lambda c, j: (0, 0)),
            pl.BlockSpec((1, C), lambda c, j: (0, 0)),
        ],
        out_specs=[
            pl.BlockSpec((1, C, BLK), lambda c, j: (c, 0, j)),
            pl.BlockSpec((1, BLK, C), lambda c, j: (c, j, 0)),
        ],
        out_shape=[
            jax.ShapeDtypeStruct((4, C, NP), jnp.float32),
            jax.ShapeDtypeStruct((4, NP, C), jnp.float32),
        ],
    )(pts_soa, f5, W8T, b.reshape(1, C))


SUBV = 625


@functools.partial(
    pl.kernel,
    mesh=_mesh,
    compiler_params=_sc_params,
    out_type=jax.ShapeDtypeStruct((4, C, HW), jnp.float32),
    scratch_types=[
        pltpu.VMEM((HW,), jnp.float32),
        pltpu.VMEM((SUBV * L,), jnp.int32),
        pltpu.VMEM((SUBV * L,), jnp.float32),
    ],
)
def _smax_kernel(flat, ptfT, dense_out, slab, flb, vb):
    cid = lax.axis_index("c")
    sid = lax.axis_index("s")
    ch = sid * 2 + cid

    def cloud_body(cloud, _):
        def z_body(j, _):
            for u in range(8):
                o = pl.multiple_of(j * (8 * L) + u * L, L)
                slab[pl.ds(o, L)] = jnp.zeros((L,), jnp.float32)
            return 0
        lax.fori_loop(0, HW // (8 * L), z_body, 0)

        def chunk_body(k, _):
            cbase = pl.multiple_of(k * (SUBV * L), L)
            pltpu.sync_copy(flat.at[cloud, pl.ds(cbase, SUBV * L)], flb)
            pltpu.sync_copy(ptfT.at[cloud, ch, pl.ds(cbase, SUBV * L)], vb)

            def v_body(j, _):
                bad = jnp.zeros((L,), jnp.int32)
                for u in range(5):
                    o = pl.multiple_of(j * (5 * L) + u * L, L)
                    iv = flb[pl.ds(o, L)]
                    vv = vb[pl.ds(o, L)]
                    cur = plsc.load_gather(slab, [iv])
                    plsc.store_scatter(slab, [iv], jnp.maximum(cur, vv))
                    re = plsc.load_gather(slab, [iv])
                    bad = bad + plsc.all_reduce_population_count(re < vv)

                @pl.when(bad[0] > 0)
                def _():
                    def fix_body(u, _):
                        o = pl.multiple_of(j * (5 * L) + u * L, L)
                        iv = flb[pl.ds(o, L)]
                        vv = vb[pl.ds(o, L)]
                        cur = plsc.load_gather(slab, [iv])

                        def cond(cur):
                            return jnp.any(cur < vv)

                        def body(cur):
                            plsc.store_scatter(slab, [iv],
                                               jnp.maximum(cur, vv),
                                               mask=cur < vv)
                            return plsc.load_gather(slab, [iv])

                        lax.while_loop(cond, body, cur)
                        return 0
                    lax.fori_loop(0, 5, fix_body, 0)
                return 0
            lax.fori_loop(0, SUBV // 5, v_body, 0)
            return 0
        lax.fori_loop(0, NV // SUBV, chunk_body, 0)

        pltpu.sync_copy(slab, dense_out.at[cloud, ch])
        return 0

    lax.fori_loop(0, 4, cloud_body, 0)


def kernel(pc0s, pc1s, W, b):
    pts_all = jnp.concatenate([pc0s, pc1s], axis=0)
    pts_soa = jnp.transpose(pts_all, (0, 2, 1))
    pts_soa = jnp.pad(pts_soa, ((0, 0), (0, 0), (0, NP - N)))
    flat, f5 = _stats_kernel(pts_soa)
    ptfT, pc0f = _matmul(pts_soa, f5, W, b)
    denseT = _smax_kernel(flat, ptfT)
    dense_4d = denseT.reshape(2, 2, C, HW).transpose(0, 1, 3, 2)
    pc0_feats = pc0f[:2, :N, :]
    return dense_4d, pc0_feats


# speedup vs baseline: 3.5794x; 1.0300x over previous
import functools

import jax
import jax.numpy as jnp
from jax import lax
from jax.experimental import pallas as pl
from jax.experimental.pallas import tpu as pltpu
from jax.experimental.pallas import tpu_sc as plsc

N = 100000
NP = 100096
HW = 256 * 256
C = 32
L = 16
TV = 391
TP = TV * L
NV = N // L
ROWS = HW // 16

_mesh = plsc.VectorSubcoreMesh(core_axis_name="c", subcore_axis_name="s",
                               num_cores=2, num_subcores=16)
_sc_params = pltpu.CompilerParams(needs_layout_passes=False,
                                  use_tc_tiling_on_sc=False)


def _flat_from_xy(x, y):
    cx = ((x + 3.2) / 0.025).astype(jnp.int32)
    cy = ((y + 3.2) / 0.025).astype(jnp.int32)
    cx = jnp.clip(cx, 0, 255)
    cy = jnp.clip(cy, 0, 255)
    return cx * 256 + cy


@functools.partial(
    pl.kernel,
    mesh=_mesh,
    compiler_params=_sc_params,
    out_type=[
        jax.ShapeDtypeStruct((4, NP), jnp.int32),
        jax.ShapeDtypeStruct((4, 5, NP), jnp.float32),
    ],
    scratch_types=[
        pltpu.VMEM((HW // 2,), jnp.float32),
        pltpu.VMEM((TP,), jnp.float32),
        pltpu.VMEM((TP,), jnp.float32),
        pltpu.VMEM((TP,), jnp.float32),
        pltpu.VMEM((TP,), jnp.int32),
        pltpu.VMEM((TP,), jnp.float32),
        pltpu.VMEM((ROWS,), jnp.float32),
        pltpu.VMEM((ROWS,), jnp.float32),
        pltpu.VMEM((ROWS,), jnp.float32),
        pltpu.VMEM((ROWS,), jnp.float32),
        pltpu.VMEM((ROWS,), jnp.float32),
        pltpu.VMEM_SHARED((8, HW // 2), jnp.float32),
    ],
)
def _stats_kernel(pts, flat_out, f5_out,
                  acc, xb, yb, zb, flb, ob, stage, cnt, sx, sy, sz, S):
    cid = lax.axis_index("c")
    sid = lax.axis_index("s")
    base = pl.multiple_of(sid * TP, L)
    valid_bound = jnp.minimum(jnp.maximum(N - sid * TP, 0), TP)
    HH = HW // 2
    myhalf = sid // 8
    lrow = pl.multiple_of((sid % 8) * ROWS, L)

    def cloud_body(i, _):
        cloud = cid * 2 + i
        pltpu.sync_copy(pts.at[cloud, 0, pl.ds(base, TP)], xb)
        pltpu.sync_copy(pts.at[cloud, 1, pl.ds(base, TP)], yb)
        pltpu.sync_copy(pts.at[cloud, 2, pl.ds(base, TP)], zb)

        def fl_body(j, _):
            o = pl.multiple_of(j * L, L)
            flb[pl.ds(o, L)] = _flat_from_xy(xb[pl.ds(o, L)], yb[pl.ds(o, L)])
            return 0
        lax.fori_loop(0, TV, fl_body, 0)

        lane = jnp.arange(L, dtype=jnp.int32)
        for comp in range(4):
            dst = (cnt, sx, sy, sz)[comp]
            for half in range(2):
                def z_body(j, _):
                    for u in range(8):
                        o = pl.multiple_of(j * (8 * L) + u * L, L)
                        acc[pl.ds(o, L)] = jnp.zeros((L,), jnp.float32)
                    return 0
                lax.fori_loop(0, HH // (8 * L), z_body, 0)

                def sc_body(j, _):
                    o = pl.multiple_of(j * L, L)
                    iv = flb[pl.ds(o, L)]
                    if comp == 0:
                        vv = jnp.ones((L,), jnp.float32)
                    elif comp == 1:
                        vv = xb[pl.ds(o, L)]
                    elif comp == 2:
                        vv = yb[pl.ds(o, L)]
                    else:
                        vv = zb[pl.ds(o, L)]
                    m = ((o + lane) < valid_bound) & ((iv >> 15) == half)
                    plsc.addupdate_scatter(acc, [iv & (HH - 1)], vv, mask=m)
                    return 0
                lax.fori_loop(0, TV, sc_body, 0)

                for rnd in range(2):
                    @pl.when((sid // 8) == rnd)
                    def _():
                        pltpu.sync_copy(acc, S.at[sid % 8])
                    plsc.subcore_barrier()

                    @pl.when(myhalf == half)
                    def _():
                        for j in range(8):
                            pltpu.sync_copy(S.at[j, pl.ds(lrow, ROWS)], stage)

                            def mg_body(t, _):
                                for u in range(4):
                                    o = pl.multiple_of(t * (4 * L) + u * L, L)
                                    if rnd == 0 and j == 0:
                                        dst[pl.ds(o, L)] = stage[pl.ds(o, L)]
                                    else:
                                        dst[pl.ds(o, L)] = (dst[pl.ds(o, L)]
                                                            + stage[pl.ds(o, L)])
                                return 0
                            lax.fori_loop(0, ROWS // (4 * L), mg_body, 0)
                    plsc.subcore_barrier()

        def mean_body(t, _):
            o = pl.multiple_of(t * L, L)
            d = jnp.maximum(cnt[pl.ds(o, L)], 1.0)
            sx[pl.ds(o, L)] = sx[pl.ds(o, L)] / d
            sy[pl.ds(o, L)] = sy[pl.ds(o, L)] / d
            sz[pl.ds(o, L)] = sz[pl.ds(o, L)] / d
            return 0
        lax.fori_loop(0, ROWS // L, mean_body, 0)
        pltpu.sync_copy(sx, S.at[0 * 2 + myhalf, pl.ds(lrow, ROWS)])
        pltpu.sync_copy(sy, S.at[1 * 2 + myhalf, pl.ds(lrow, ROWS)])
        pltpu.sync_copy(sz, S.at[2 * 2 + myhalf, pl.ds(lrow, ROWS)])
        plsc.subcore_barrier()

        for comp in range(3):
            src = (xb, yb, zb)[comp]
            for half in range(2):
                pltpu.sync_copy(S.at[comp * 2 + half], acc)

                def g_body(j, _):
                    o = pl.multiple_of(j * L, L)
                    iv = flb[pl.ds(o, L)]
                    g = plsc.load_gather(acc, [iv & (HH - 1)])
                    r = src[pl.ds(o, L)] - g
                    if half == 0:
                        ob[pl.ds(o, L)] = r
                    else:
                        keep = (iv >> 15) == 1
                        ob[pl.ds(o, L)] = jnp.where(keep, r, ob[pl.ds(o, L)])
                    return 0
                lax.fori_loop(0, TV, g_body, 0)
            pltpu.sync_copy(ob, f5_out.at[cloud, comp, pl.ds(base, TP)])

        def fc_body(j, _):
            o = pl.multiple_of(j * L, L)
            fl = flb[pl.ds(o, L)]
            cxf = (fl >> 8).astype(jnp.float32)
            ob[pl.ds(o, L)] = xb[pl.ds(o, L)] - ((cxf + 0.5) * 0.025 + (-3.2))
            return 0
        lax.fori_loop(0, TV, fc_body, 0)
        pltpu.sync_copy(ob, f5_out.at[cloud, 3, pl.ds(base, TP)])

        def fcy_body(j, _):
            o = pl.multiple_of(j * L, L)
            fl = flb[pl.ds(o, L)]
            cyf = (fl & 255).astype(jnp.float32)
            ob[pl.ds(o, L)] = yb[pl.ds(o, L)] - ((cyf + 0.5) * 0.025 + (-3.2))
            return 0
        lax.fori_loop(0, TV, fcy_body, 0)
        pltpu.sync_copy(ob, f5_out.at[cloud, 4, pl.ds(base, TP)])

        pltpu.sync_copy(flb, flat_out.at[cloud, pl.ds(base, TP)])
        plsc.subcore_barrier()
        return 0

    lax.fori_loop(0, 2, cloud_body, 0)


BLK = 8192


def _mm_kernel(pts_ref, f5_ref, w_ref, b_ref, ptfT_ref, pc0_ref):
    x3 = pts_ref[0]
    f5 = f5_ref[0]
    x8 = jnp.concatenate([x3, f5], axis=0)
    w = w_ref[...]
    out = jnp.broadcast_to(b_ref[...].reshape(C, 1), (C, x8.shape[1]))
    for k in range(8):
        out = out + w[:, k:k + 1] * x8[k:k + 1, :]
    out = jnp.maximum(out, 0.0)
    ptfT_ref[0] = out
    pc0_ref[0] = out.T


def _matmul(pts_soa, f5, W, b):
    W8 = jnp.concatenate(
        [W[0:2], (W[2:3] + W[8:9]), W[3:8]], axis=0)
    W8T = W8.T
    nb = (NP + BLK - 1) // BLK
    return pl.pallas_call(
        _mm_kernel,
        grid=(4, nb),
        in_specs=[
            pl.BlockSpec((1, 3, BLK), lambda c, j: (c, 0, j)),
            pl.BlockSpec((1, 5, BLK), lambda c, j: (c, 0, j)),
            pl.BlockSpec((C, 8), lambda c, j: (0, 0)),
            pl.BlockSpec((1, C), lambda c, j: (0, 0)),
        ],
        out_specs=[
            pl.BlockSpec((1, C, BLK), lambda c, j: (c, 0, j)),
            pl.BlockSpec((1, BLK, C), lambda c, j: (c, j, 0)),
        ],
        out_shape=[
            jax.ShapeDtypeStruct((4, C, NP), jnp.float32),
            jax.ShapeDtypeStruct((4, NP, C), jnp.float32),
        ],
    )(pts_soa, f5, W8T, b.reshape(1, C))


SUBV = 625


@functools.partial(
    pl.kernel,
    mesh=_mesh,
    compiler_params=_sc_params,
    out_type=jax.ShapeDtypeStruct((4, C, HW), jnp.float32),
    scratch_types=[
        pltpu.VMEM((HW,), jnp.float32),
        pltpu.VMEM((SUBV * L,), jnp.int32),
        pltpu.VMEM((SUBV * L,), jnp.float32),
        pltpu.SemaphoreType.DMA,
    ],
)
def _smax_kernel(flat, ptfT, dense_out, slab, flb, vb, sem):
    cid = lax.axis_index("c")
    sid = lax.axis_index("s")
    ch = sid * 2 + cid

    def cloud_body(cloud, _):
        def z_body(j, _):
            for u in range(8):
                o = pl.multiple_of(j * (8 * L) + u * L, L)
                slab[pl.ds(o, L)] = jnp.zeros((L,), jnp.float32)
            return 0
        lax.fori_loop(0, HW // (8 * L), z_body, 0)

        def chunk_body(k, _):
            cbase = pl.multiple_of(k * (SUBV * L), L)
            c1 = pltpu.async_copy(flat.at[cloud, pl.ds(cbase, SUBV * L)], flb, sem)
            c2 = pltpu.async_copy(ptfT.at[cloud, ch, pl.ds(cbase, SUBV * L)], vb, sem)
            c1.wait()
            c2.wait()

            def v_body(j, _):
                bad = jnp.zeros((L,), jnp.int32)
                for u in range(5):
                    o = pl.multiple_of(j * (5 * L) + u * L, L)
                    iv = flb[pl.ds(o, L)]
                    vv = vb[pl.ds(o, L)]
                    _, lastm = plsc.scan_count(iv)
                    cur = plsc.load_gather(slab, [iv])
                    plsc.store_scatter(slab, [iv], jnp.maximum(cur, vv))
                    bad = bad + (16 - plsc.all_reduce_population_count(lastm))

                @pl.when(bad[0] > 0)
                def _():
                    def fix_body(u, _):
                        o = pl.multiple_of(j * (5 * L) + u * L, L)
                        iv = flb[pl.ds(o, L)]
                        vv = vb[pl.ds(o, L)]
                        cur = plsc.load_gather(slab, [iv])

                        def cond(cur):
                            return jnp.any(cur < vv)

                        def body(cur):
                            plsc.store_scatter(slab, [iv],
                                               jnp.maximum(cur, vv),
                                               mask=cur < vv)
                            return plsc.load_gather(slab, [iv])

                        lax.while_loop(cond, body, cur)
                        return 0
                    lax.fori_loop(0, 5, fix_body, 0)
                return 0
            lax.fori_loop(0, SUBV // 5, v_body, 0)
            return 0
        lax.fori_loop(0, NV // SUBV, chunk_body, 0)

        pltpu.sync_copy(slab, dense_out.at[cloud, ch])
        return 0

    lax.fori_loop(0, 4, cloud_body, 0)


def kernel(pc0s, pc1s, W, b):
    pts_all = jnp.concatenate([pc0s, pc1s], axis=0)
    pts_soa = jnp.transpose(pts_all, (0, 2, 1))
    pts_soa = jnp.pad(pts_soa, ((0, 0), (0, 0), (0, NP - N)))
    flat, f5 = _stats_kernel(pts_soa)
    ptfT, pc0f = _matmul(pts_soa, f5, W, b)
    denseT = _smax_kernel(flat, ptfT)
    dense_4d = denseT.reshape(2, 2, C, HW).transpose(0, 1, 3, 2)
    pc0_feats = pc0f[:2, :N, :]
    return dense_4d, pc0_feats


# speedup vs baseline: 3.6122x; 1.0092x over previous
import functools

import jax
import jax.numpy as jnp
from jax import lax
from jax.experimental import pallas as pl
from jax.experimental.pallas import tpu as pltpu
from jax.experimental.pallas import tpu_sc as plsc

N = 100000
NP = 100096
HW = 256 * 256
C = 32
L = 16
TV = 391
TP = TV * L
NV = N // L
ROWS = HW // 16

_mesh = plsc.VectorSubcoreMesh(core_axis_name="c", subcore_axis_name="s",
                               num_cores=2, num_subcores=16)
_sc_params = pltpu.CompilerParams(needs_layout_passes=False,
                                  use_tc_tiling_on_sc=False)


def _flat_from_xy(x, y):
    cx = ((x + 3.2) / 0.025).astype(jnp.int32)
    cy = ((y + 3.2) / 0.025).astype(jnp.int32)
    cx = jnp.clip(cx, 0, 255)
    cy = jnp.clip(cy, 0, 255)
    return cx * 256 + cy


@functools.partial(
    pl.kernel,
    mesh=_mesh,
    compiler_params=_sc_params,
    out_type=[
        jax.ShapeDtypeStruct((4, NP), jnp.int32),
        jax.ShapeDtypeStruct((4, 5, NP), jnp.float32),
        jax.ShapeDtypeStruct((4, 16, 32), jnp.int32),
    ],
    scratch_types=[
        pltpu.VMEM((HW // 2,), jnp.float32),
        pltpu.VMEM((TP,), jnp.float32),
        pltpu.VMEM((TP,), jnp.float32),
        pltpu.VMEM((TP,), jnp.float32),
        pltpu.VMEM((TP,), jnp.int32),
        pltpu.VMEM((TP,), jnp.float32),
        pltpu.VMEM((ROWS,), jnp.float32),
        pltpu.VMEM((ROWS,), jnp.float32),
        pltpu.VMEM((ROWS,), jnp.float32),
        pltpu.VMEM((ROWS,), jnp.float32),
        pltpu.VMEM((ROWS,), jnp.float32),
        pltpu.VMEM((32,), jnp.int32),
        pltpu.VMEM_SHARED((8, HW // 2), jnp.float32),
    ],
)
def _stats_kernel(pts, flat_out, f5_out, flags_out,
                  acc, xb, yb, zb, flb, ob, stage, cnt, sx, sy, sz, flagbuf, S):
    cid = lax.axis_index("c")
    sid = lax.axis_index("s")
    base = pl.multiple_of(sid * TP, L)
    valid_bound = jnp.minimum(jnp.maximum(N - sid * TP, 0), TP)
    HH = HW // 2
    myhalf = sid // 8
    lrow = pl.multiple_of((sid % 8) * ROWS, L)

    def cloud_body(i, _):
        cloud = cid * 2 + i
        pltpu.sync_copy(pts.at[cloud, 0, pl.ds(base, TP)], xb)
        pltpu.sync_copy(pts.at[cloud, 1, pl.ds(base, TP)], yb)
        pltpu.sync_copy(pts.at[cloud, 2, pl.ds(base, TP)], zb)

        lane = jnp.arange(L, dtype=jnp.int32)

        def do_vreg(j):
            o = pl.multiple_of(j * L, L)
            fl = _flat_from_xy(xb[pl.ds(o, L)], yb[pl.ds(o, L)])
            m = (o + lane) < valid_bound
            fl = jnp.where(m, fl, HW)
            flb[pl.ds(o, L)] = fl
            _, lastm = plsc.scan_count(fl)
            return 16 - plsc.all_reduce_population_count(lastm)

        flagbuf[pl.ds(16, L)] = jnp.zeros((L,), jnp.int32)

        def win_body(g, _):
            acc16 = jnp.zeros((L,), jnp.int32)
            for u in range(16):
                acc16 = acc16 + do_vreg(g * 16 + u)
            plsc.store_scatter(flagbuf, [jnp.full((L,), 0, jnp.int32) + g],
                               acc16, mask=lane == 0)
            return 0
        lax.fori_loop(0, TV // 16, win_body, 0)
        acc16 = jnp.zeros((L,), jnp.int32)
        for u in range(TV % 16):
            acc16 = acc16 + do_vreg((TV // 16) * 16 + u)
        plsc.store_scatter(flagbuf, [jnp.full((L,), TV // 16, jnp.int32)],
                           acc16, mask=lane == 0)
        pltpu.sync_copy(flagbuf, flags_out.at[cloud, sid])

        for comp in range(4):
            dst = (cnt, sx, sy, sz)[comp]
            for half in range(2):
                def z_body(j, _):
                    for u in range(8):
                        o = pl.multiple_of(j * (8 * L) + u * L, L)
                        acc[pl.ds(o, L)] = jnp.zeros((L,), jnp.float32)
                    return 0
                lax.fori_loop(0, HH // (8 * L), z_body, 0)

                def sc_body(j, _):
                    o = pl.multiple_of(j * L, L)
                    iv = flb[pl.ds(o, L)]
                    if comp == 0:
                        vv = jnp.ones((L,), jnp.float32)
                    elif comp == 1:
                        vv = xb[pl.ds(o, L)]
                    elif comp == 2:
                        vv = yb[pl.ds(o, L)]
                    else:
                        vv = zb[pl.ds(o, L)]
                    m = (iv >> 15) == half
                    plsc.addupdate_scatter(acc, [iv & (HH - 1)], vv, mask=m)
                    return 0
                lax.fori_loop(0, TV, sc_body, 0)

                for rnd in range(2):
                    @pl.when((sid // 8) == rnd)
                    def _():
                        pltpu.sync_copy(acc, S.at[sid % 8])
                    plsc.subcore_barrier()

                    @pl.when(myhalf == half)
                    def _():
                        for j in range(8):
                            pltpu.sync_copy(S.at[j, pl.ds(lrow, ROWS)], stage)

                            def mg_body(t, _):
                                for u in range(4):
                                    o = pl.multiple_of(t * (4 * L) + u * L, L)
                                    if rnd == 0 and j == 0:
                                        dst[pl.ds(o, L)] = stage[pl.ds(o, L)]
                                    else:
                                        dst[pl.ds(o, L)] = (dst[pl.ds(o, L)]
                                                            + stage[pl.ds(o, L)])
                                return 0
                            lax.fori_loop(0, ROWS // (4 * L), mg_body, 0)
                    plsc.subcore_barrier()

        def mean_body(t, _):
            o = pl.multiple_of(t * L, L)
            d = jnp.maximum(cnt[pl.ds(o, L)], 1.0)
            sx[pl.ds(o, L)] = sx[pl.ds(o, L)] / d
            sy[pl.ds(o, L)] = sy[pl.ds(o, L)] / d
            sz[pl.ds(o, L)] = sz[pl.ds(o, L)] / d
            return 0
        lax.fori_loop(0, ROWS // L, mean_body, 0)
        pltpu.sync_copy(sx, S.at[0 * 2 + myhalf, pl.ds(lrow, ROWS)])
        pltpu.sync_copy(sy, S.at[1 * 2 + myhalf, pl.ds(lrow, ROWS)])
        pltpu.sync_copy(sz, S.at[2 * 2 + myhalf, pl.ds(lrow, ROWS)])
        plsc.subcore_barrier()

        for comp in range(3):
            src = (xb, yb, zb)[comp]
            for half in range(2):
                pltpu.sync_copy(S.at[comp * 2 + half], acc)

                def g_body(j, _):
                    o = pl.multiple_of(j * L, L)
                    iv = flb[pl.ds(o, L)]
                    g = plsc.load_gather(acc, [iv & (HH - 1)])
                    r = src[pl.ds(o, L)] - g
                    if half == 0:
                        ob[pl.ds(o, L)] = r
                    else:
                        keep = (iv >> 15) == 1
                        ob[pl.ds(o, L)] = jnp.where(keep, r, ob[pl.ds(o, L)])
                    return 0
                lax.fori_loop(0, TV, g_body, 0)
            pltpu.sync_copy(ob, f5_out.at[cloud, comp, pl.ds(base, TP)])

        def fc_body(j, _):
            o = pl.multiple_of(j * L, L)
            fl = flb[pl.ds(o, L)]
            cxf = (fl >> 8).astype(jnp.float32)
            ob[pl.ds(o, L)] = xb[pl.ds(o, L)] - ((cxf + 0.5) * 0.025 + (-3.2))
            return 0
        lax.fori_loop(0, TV, fc_body, 0)
        pltpu.sync_copy(ob, f5_out.at[cloud, 3, pl.ds(base, TP)])

        def fcy_body(j, _):
            o = pl.multiple_of(j * L, L)
            fl = flb[pl.ds(o, L)]
            cyf = (fl & 255).astype(jnp.float32)
            ob[pl.ds(o, L)] = yb[pl.ds(o, L)] - ((cyf + 0.5) * 0.025 + (-3.2))
            return 0
        lax.fori_loop(0, TV, fcy_body, 0)
        pltpu.sync_copy(ob, f5_out.at[cloud, 4, pl.ds(base, TP)])

        pltpu.sync_copy(flb, flat_out.at[cloud, pl.ds(base, TP)])
        plsc.subcore_barrier()
        return 0

    lax.fori_loop(0, 2, cloud_body, 0)


BLK = 8192


def _mm_kernel(pts_ref, f5_ref, w_ref, b_ref, ptfT_ref, pc0_ref):
    x3 = pts_ref[0]
    f5 = f5_ref[0]
    x8 = jnp.concatenate([x3, f5], axis=0)
    w = w_ref[...]
    out = jnp.broadcast_to(b_ref[...].reshape(C, 1), (C, x8.shape[1]))
    for k in range(8):
        out = out + w[:, k:k + 1] * x8[k:k + 1, :]
    out = jnp.maximum(out, 0.0)
    ptfT_ref[0] = out
    pc0_ref[0] = out.T


def _matmul(pts_soa, f5, W, b):
    W8 = jnp.concatenate(
        [W[0:2], (W[2:3] + W[8:9]), W[3:8]], axis=0)
    W8T = W8.T
    nb = (NP + BLK - 1) // BLK
    return pl.pallas_call(
        _mm_kernel,
        grid=(4, nb),
        in_specs=[
            pl.BlockSpec((1, 3, BLK), lambda c, j: (c, 0, j)),
            pl.BlockSpec((1, 5, BLK), lambda c, j: (c, 0, j)),
            pl.BlockSpec((C, 8), lambda c, j: (0, 0)),
            pl.BlockSpec((1, C), lambda c, j: (0, 0)),
        ],
        out_specs=[
            pl.BlockSpec((1, C, BLK), lambda c, j: (c, 0, j)),
            pl.BlockSpec((1, BLK, C), lambda c, j: (c, j, 0)),
        ],
        out_shape=[
            jax.ShapeDtypeStruct((4, C, NP), jnp.float32),
            jax.ShapeDtypeStruct((4, NP, C), jnp.float32),
        ],
    )(pts_soa, f5, W8T, b.reshape(1, C))


SLAB = HW + L


@functools.partial(
    pl.kernel,
    mesh=_mesh,
    compiler_params=_sc_params,
    out_type=jax.ShapeDtypeStruct((4, C, HW), jnp.float32),
    scratch_types=[
        pltpu.VMEM((SLAB,), jnp.float32),
        pltpu.VMEM((TP,), jnp.int32),
        pltpu.VMEM((TP,), jnp.float32),
        pltpu.VMEM((32,), jnp.int32),
        pltpu.SemaphoreType.DMA,
    ],
)
def _smax_kernel(flat, ptfT, flags, dense_out, slab, flb, vb, flagv, sem):
    cid = lax.axis_index("c")
    sid = lax.axis_index("s")
    ch = sid * 2 + cid

    def fix_vreg(o):
        iv = flb[pl.ds(o, L)]
        vv = vb[pl.ds(o, L)]
        cur = plsc.load_gather(slab, [iv])

        def cond(cur):
            return jnp.any(cur < vv)

        def body(cur):
            plsc.store_scatter(slab, [iv], jnp.maximum(cur, vv), mask=cur < vv)
            return plsc.load_gather(slab, [iv])

        lax.while_loop(cond, body, cur)

    def rmw_vreg(o):
        iv = flb[pl.ds(o, L)]
        vv = vb[pl.ds(o, L)]
        cur = plsc.load_gather(slab, [iv])
        plsc.store_scatter(slab, [iv], jnp.maximum(cur, vv))

    def cloud_body(cloud, _):
        def z_body(j, _):
            for u in range(8):
                o = pl.multiple_of(j * (8 * L) + u * L, L)
                slab[pl.ds(o, L)] = jnp.zeros((L,), jnp.float32)
            return 0
        lax.fori_loop(0, SLAB // (8 * L), z_body, 0)

        def chunk_body(t, _):
            cbase = pl.multiple_of(t * TP, L)
            c1 = pltpu.async_copy(flat.at[cloud, pl.ds(cbase, TP)], flb, sem)
            c2 = pltpu.async_copy(ptfT.at[cloud, ch, pl.ds(cbase, TP)], vb, sem)
            c3 = pltpu.async_copy(flags.at[cloud, t], flagv, sem)
            c1.wait()
            c2.wait()
            c3.wait()

            def v_body(g, _):
                for u in range(16):
                    rmw_vreg(pl.multiple_of((g * 16 + u) * L, L))
                fg = plsc.load_gather(flagv, [jnp.zeros((L,), jnp.int32) + g])

                @pl.when(fg[0] > 0)
                def _():
                    def fix_body(u, _):
                        fix_vreg(pl.multiple_of((g * 16 + u) * L, L))
                        return 0
                    lax.fori_loop(0, 16, fix_body, 0)
                return 0
            lax.fori_loop(0, TV // 16, v_body, 0)
            for u in range(TV % 16):
                rmw_vreg(((TV // 16) * 16 + u) * L)
            fg = plsc.load_gather(flagv, [jnp.full((L,), TV // 16, jnp.int32)])

            @pl.when(fg[0] > 0)
            def _():
                def fix_body(u, _):
                    fix_vreg(pl.multiple_of(((TV // 16) * 16 + u) * L, L))
                    return 0
                lax.fori_loop(0, TV % 16, fix_body, 0)
            return 0
        lax.fori_loop(0, 16, chunk_body, 0)

        pltpu.sync_copy(slab.at[pl.ds(0, HW)], dense_out.at[cloud, ch])
        return 0

    lax.fori_loop(0, 4, cloud_body, 0)


def kernel(pc0s, pc1s, W, b):
    pts_all = jnp.concatenate([pc0s, pc1s], axis=0)
    pts_soa = jnp.transpose(pts_all, (0, 2, 1))
    pts_soa = jnp.pad(pts_soa, ((0, 0), (0, 0), (0, NP - N)))
    flat, f5, flags = _stats_kernel(pts_soa)
    ptfT, pc0f = _matmul(pts_soa, f5, W, b)
    denseT = _smax_kernel(flat, ptfT, flags)
    dense_4d = denseT.reshape(2, 2, C, HW).transpose(0, 1, 3, 2)
    pc0_feats = pc0f[:2, :N, :]
    return dense_4d, pc0_feats


# speedup vs baseline: 3.9367x; 1.0898x over previous
import functools

import jax
import jax.numpy as jnp
from jax import lax
from jax.experimental import pallas as pl
from jax.experimental.pallas import tpu as pltpu
from jax.experimental.pallas import tpu_sc as plsc

N = 100000
NP = 100096
HW = 256 * 256
C = 32
L = 16
TV = 391
TP = TV * L
NV = N // L
ROWS = HW // 16

_mesh = plsc.VectorSubcoreMesh(core_axis_name="c", subcore_axis_name="s",
                               num_cores=2, num_subcores=16)
_sc_params = pltpu.CompilerParams(needs_layout_passes=False,
                                  use_tc_tiling_on_sc=False)


def _flat_from_xy(x, y):
    cx = ((x + 3.2) / 0.025).astype(jnp.int32)
    cy = ((y + 3.2) / 0.025).astype(jnp.int32)
    cx = jnp.clip(cx, 0, 255)
    cy = jnp.clip(cy, 0, 255)
    return cx * 256 + cy


@functools.partial(
    pl.kernel,
    mesh=_mesh,
    compiler_params=_sc_params,
    out_type=[
        jax.ShapeDtypeStruct((4, NP), jnp.int32),
        jax.ShapeDtypeStruct((4, 5, NP), jnp.float32),
        jax.ShapeDtypeStruct((4, 16, 32), jnp.int32),
    ],
    scratch_types=[
        pltpu.VMEM((HW // 2,), jnp.float32),
        pltpu.VMEM((TP,), jnp.float32),
        pltpu.VMEM((TP,), jnp.float32),
        pltpu.VMEM((TP,), jnp.float32),
        pltpu.VMEM((TP,), jnp.int32),
        pltpu.VMEM((TP,), jnp.float32),
        pltpu.VMEM((ROWS,), jnp.float32),
        pltpu.VMEM((ROWS,), jnp.float32),
        pltpu.VMEM((ROWS,), jnp.float32),
        pltpu.VMEM((ROWS,), jnp.float32),
        pltpu.VMEM((ROWS,), jnp.float32),
        pltpu.VMEM((32,), jnp.int32),
        pltpu.VMEM_SHARED((8, HW // 2), jnp.float32),
    ],
)
def _stats_kernel(pts, flat_out, f5_out, flags_out,
                  acc, xb, yb, zb, flb, ob, stage, cnt, sx, sy, sz, flagbuf, S):
    cid = lax.axis_index("c")
    sid = lax.axis_index("s")
    base = pl.multiple_of(sid * TP, L)
    valid_bound = jnp.minimum(jnp.maximum(N - sid * TP, 0), TP)
    HH = HW // 2
    myhalf = sid // 8
    lrow = pl.multiple_of((sid % 8) * ROWS, L)

    def cloud_body(i, _):
        cloud = cid * 2 + i
        pltpu.sync_copy(pts.at[cloud, 0, pl.ds(base, TP)], xb)
        pltpu.sync_copy(pts.at[cloud, 1, pl.ds(base, TP)], yb)
        pltpu.sync_copy(pts.at[cloud, 2, pl.ds(base, TP)], zb)

        lane = jnp.arange(L, dtype=jnp.int32)

        def do_vreg(j):
            o = pl.multiple_of(j * L, L)
            fl = _flat_from_xy(xb[pl.ds(o, L)], yb[pl.ds(o, L)])
            m = (o + lane) < valid_bound
            fl = jnp.where(m, fl, HW)
            flb[pl.ds(o, L)] = fl
            _, lastm = plsc.scan_count(fl)
            return 16 - plsc.all_reduce_population_count(lastm)

        flagbuf[pl.ds(16, L)] = jnp.zeros((L,), jnp.int32)

        def win_body(g, _):
            acc16 = jnp.zeros((L,), jnp.int32)
            for u in range(16):
                acc16 = acc16 + do_vreg(g * 16 + u)
            plsc.store_scatter(flagbuf, [jnp.full((L,), 0, jnp.int32) + g],
                               acc16, mask=lane == 0)
            return 0
        lax.fori_loop(0, TV // 16, win_body, 0)
        acc16 = jnp.zeros((L,), jnp.int32)
        for u in range(TV % 16):
            acc16 = acc16 + do_vreg((TV // 16) * 16 + u)
        plsc.store_scatter(flagbuf, [jnp.full((L,), TV // 16, jnp.int32)],
                           acc16, mask=lane == 0)
        pltpu.sync_copy(flagbuf, flags_out.at[cloud, sid])

        for comp in range(4):
            dst = (cnt, sx, sy, sz)[comp]
            for half in range(2):
                def z_body(j, _):
                    for u in range(8):
                        o = pl.multiple_of(j * (8 * L) + u * L, L)
                        acc[pl.ds(o, L)] = jnp.zeros((L,), jnp.float32)
                    return 0
                lax.fori_loop(0, HH // (8 * L), z_body, 0)

                def sc_body(j, _):
                    o = pl.multiple_of(j * L, L)
                    iv = flb[pl.ds(o, L)]
                    if comp == 0:
                        vv = jnp.ones((L,), jnp.float32)
                    elif comp == 1:
                        vv = xb[pl.ds(o, L)]
                    elif comp == 2:
                        vv = yb[pl.ds(o, L)]
                    else:
                        vv = zb[pl.ds(o, L)]
                    m = (iv >> 15) == half
                    plsc.addupdate_scatter(acc, [iv & (HH - 1)], vv, mask=m)
                    return 0
                lax.fori_loop(0, TV, sc_body, 0)

                for rnd in range(2):
                    @pl.when((sid // 8) == rnd)
                    def _():
                        pltpu.sync_copy(acc, S.at[sid % 8])
                    plsc.subcore_barrier()

                    @pl.when(myhalf == half)
                    def _():
                        for j in range(8):
                            pltpu.sync_copy(S.at[j, pl.ds(lrow, ROWS)], stage)

                            def mg_body(t, _):
                                for u in range(4):
                                    o = pl.multiple_of(t * (4 * L) + u * L, L)
                                    if rnd == 0 and j == 0:
                                        dst[pl.ds(o, L)] = stage[pl.ds(o, L)]
                                    else:
                                        dst[pl.ds(o, L)] = (dst[pl.ds(o, L)]
                                                            + stage[pl.ds(o, L)])
                                return 0
                            lax.fori_loop(0, ROWS // (4 * L), mg_body, 0)
                    plsc.subcore_barrier()

        def mean_body(t, _):
            o = pl.multiple_of(t * L, L)
            d = jnp.maximum(cnt[pl.ds(o, L)], 1.0)
            sx[pl.ds(o, L)] = sx[pl.ds(o, L)] / d
            sy[pl.ds(o, L)] = sy[pl.ds(o, L)] / d
            sz[pl.ds(o, L)] = sz[pl.ds(o, L)] / d
            return 0
        lax.fori_loop(0, ROWS // L, mean_body, 0)
        pltpu.sync_copy(sx, S.at[0 * 2 + myhalf, pl.ds(lrow, ROWS)])
        pltpu.sync_copy(sy, S.at[1 * 2 + myhalf, pl.ds(lrow, ROWS)])
        pltpu.sync_copy(sz, S.at[2 * 2 + myhalf, pl.ds(lrow, ROWS)])
        plsc.subcore_barrier()

        for comp in range(3):
            src = (xb, yb, zb)[comp]
            for half in range(2):
                pltpu.sync_copy(S.at[comp * 2 + half], acc)

                def g_body(j, _):
                    o = pl.multiple_of(j * L, L)
                    iv = flb[pl.ds(o, L)]
                    g = plsc.load_gather(acc, [iv & (HH - 1)])
                    r = src[pl.ds(o, L)] - g
                    if half == 0:
                        ob[pl.ds(o, L)] = r
                    else:
                        keep = (iv >> 15) == 1
                        ob[pl.ds(o, L)] = jnp.where(keep, r, ob[pl.ds(o, L)])
                    return 0
                lax.fori_loop(0, TV, g_body, 0)
            pltpu.sync_copy(ob, f5_out.at[cloud, comp, pl.ds(base, TP)])

        def fc_body(j, _):
            o = pl.multiple_of(j * L, L)
            fl = flb[pl.ds(o, L)]
            cxf = (fl >> 8).astype(jnp.float32)
            ob[pl.ds(o, L)] = xb[pl.ds(o, L)] - ((cxf + 0.5) * 0.025 + (-3.2))
            return 0
        lax.fori_loop(0, TV, fc_body, 0)
        pltpu.sync_copy(ob, f5_out.at[cloud, 3, pl.ds(base, TP)])

        def fcy_body(j, _):
            o = pl.multiple_of(j * L, L)
            fl = flb[pl.ds(o, L)]
            cyf = (fl & 255).astype(jnp.float32)
            ob[pl.ds(o, L)] = yb[pl.ds(o, L)] - ((cyf + 0.5) * 0.025 + (-3.2))
            return 0
        lax.fori_loop(0, TV, fcy_body, 0)
        pltpu.sync_copy(ob, f5_out.at[cloud, 4, pl.ds(base, TP)])

        pltpu.sync_copy(flb, flat_out.at[cloud, pl.ds(base, TP)])
        plsc.subcore_barrier()
        return 0

    lax.fori_loop(0, 2, cloud_body, 0)


BLK = 8192


def _mm_kernel(pts_ref, f5_ref, w_ref, b_ref, ptfT_ref, pc0_ref):
    x3 = pts_ref[0]
    f5 = f5_ref[0]
    x8 = jnp.concatenate([x3, f5], axis=0)
    w = w_ref[...]
    out = jnp.broadcast_to(b_ref[...].reshape(C, 1), (C, x8.shape[1]))
    for k in range(8):
        out = out + w[:, k:k + 1] * x8[k:k + 1, :]
    out = jnp.maximum(out, 0.0)
    ptfT_ref[0] = out
    pc0_ref[0] = out.T


def _matmul(pts_soa, f5, W, b):
    W8 = jnp.concatenate(
        [W[0:2], (W[2:3] + W[8:9]), W[3:8]], axis=0)
    W8T = W8.T
    nb = (NP + BLK - 1) // BLK
    return pl.pallas_call(
        _mm_kernel,
        grid=(4, nb),
        in_specs=[
            pl.BlockSpec((1, 3, BLK), lambda c, j: (c, 0, j)),
            pl.BlockSpec((1, 5, BLK), lambda c, j: (c, 0, j)),
            pl.BlockSpec((C, 8), lambda c, j: (0, 0)),
            pl.BlockSpec((1, C), lambda c, j: (0, 0)),
        ],
        out_specs=[
            pl.BlockSpec((1, C, BLK), lambda c, j: (c, 0, j)),
            pl.BlockSpec((1, BLK, C), lambda c, j: (c, j, 0)),
        ],
        out_shape=[
            jax.ShapeDtypeStruct((4, C, NP), jnp.float32),
            jax.ShapeDtypeStruct((4, NP, C), jnp.float32),
        ],
    )(pts_soa, f5, W8T, b.reshape(1, C))


SLAB = HW + L


@functools.partial(
    pl.kernel,
    mesh=_mesh,
    compiler_params=_sc_params,
    out_type=jax.ShapeDtypeStruct((4, C, HW), jnp.float32),
    scratch_types=[
        pltpu.VMEM((SLAB,), jnp.float32),
        pltpu.VMEM((TP,), jnp.int32),
        pltpu.VMEM((TP,), jnp.int32),
        pltpu.VMEM((TP,), jnp.float32),
        pltpu.VMEM((TP,), jnp.float32),
        pltpu.VMEM((32,), jnp.int32),
        pltpu.VMEM((32,), jnp.int32),
        pltpu.SemaphoreType.DMA,
        pltpu.SemaphoreType.DMA,
    ],
)
def _smax_kernel(flat, ptfT, flags, dense_out, slab,
                 flb0, flb1, vb0, vb1, fg0, fg1, sem0, sem1):
    cid = lax.axis_index("c")
    sid = lax.axis_index("s")
    ch = sid * 2 + cid
    bufs = ((flb0, vb0, fg0, sem0), (flb1, vb1, fg1, sem1))

    def issue(cloud, t, b):
        flb, vb, fgv, sem = bufs[b]
        cbase = pl.multiple_of(t * TP, L)
        pltpu.async_copy(flat.at[cloud, pl.ds(cbase, TP)], flb, sem)
        pltpu.async_copy(ptfT.at[cloud, ch, pl.ds(cbase, TP)], vb, sem)
        pltpu.async_copy(flags.at[cloud, t], fgv, sem)

    def drain(cloud, t, b):
        flb, vb, fgv, sem = bufs[b]
        cbase = pl.multiple_of(t * TP, L)
        pltpu.make_async_copy(flat.at[cloud, pl.ds(cbase, TP)], flb, sem).wait()
        pltpu.make_async_copy(ptfT.at[cloud, ch, pl.ds(cbase, TP)], vb, sem).wait()
        pltpu.make_async_copy(flags.at[cloud, t], fgv, sem).wait()

    def fix_vreg(flb, vb, o):
        iv = flb[pl.ds(o, L)]
        vv = vb[pl.ds(o, L)]
        cur = plsc.load_gather(slab, [iv])

        def cond(cur):
            return jnp.any(cur < vv)

        def body(cur):
            plsc.store_scatter(slab, [iv], jnp.maximum(cur, vv), mask=cur < vv)
            return plsc.load_gather(slab, [iv])

        lax.while_loop(cond, body, cur)

    def rmw_vreg(flb, vb, o):
        iv = flb[pl.ds(o, L)]
        vv = vb[pl.ds(o, L)]
        cur = plsc.load_gather(slab, [iv])
        plsc.store_scatter(slab, [iv], jnp.maximum(cur, vv))

    def process(b):
        flb, vb, fgv, _ = bufs[b]

        def v_body(g, _):
            for u in range(16):
                rmw_vreg(flb, vb, pl.multiple_of((g * 16 + u) * L, L))
            fg = plsc.load_gather(fgv, [jnp.zeros((L,), jnp.int32) + g])

            @pl.when(fg[0] > 0)
            def _():
                def fix_body(u, _):
                    fix_vreg(flb, vb, pl.multiple_of((g * 16 + u) * L, L))
                    return 0
                lax.fori_loop(0, 16, fix_body, 0)
            return 0
        lax.fori_loop(0, TV // 16, v_body, 0)
        for u in range(TV % 16):
            rmw_vreg(flb, vb, ((TV // 16) * 16 + u) * L)
        fg = plsc.load_gather(fgv, [jnp.full((L,), TV // 16, jnp.int32)])

        @pl.when(fg[0] > 0)
        def _():
            def fix_body(u, _):
                fix_vreg(flb, vb, pl.multiple_of(((TV // 16) * 16 + u) * L, L))
                return 0
            lax.fori_loop(0, TV % 16, fix_body, 0)

    def cloud_body(cloud, _):
        def z_body(j, _):
            for u in range(8):
                o = pl.multiple_of(j * (8 * L) + u * L, L)
                slab[pl.ds(o, L)] = jnp.zeros((L,), jnp.float32)
            return 0
        lax.fori_loop(0, SLAB // (8 * L), z_body, 0)

        issue(cloud, 0, 0)

        def pair_body(p, _):
            for b in range(2):
                t = p * 2 + b
                drain(cloud, t, b)

                @pl.when(t + 1 < 16)
                def _():
                    issue(cloud, t + 1, 1 - b)
                process(b)
            return 0
        lax.fori_loop(0, 8, pair_body, 0)

        pltpu.sync_copy(slab.at[pl.ds(0, HW)], dense_out.at[cloud, ch])
        return 0

    lax.fori_loop(0, 4, cloud_body, 0)


def kernel(pc0s, pc1s, W, b):
    pts_all = jnp.concatenate([pc0s, pc1s], axis=0)
    pts_soa = jnp.transpose(pts_all, (0, 2, 1))
    pts_soa = jnp.pad(pts_soa, ((0, 0), (0, 0), (0, NP - N)))
    flat, f5, flags = _stats_kernel(pts_soa)
    ptfT, pc0f = _matmul(pts_soa, f5, W, b)
    denseT = _smax_kernel(flat, ptfT, flags)
    dense_4d = denseT.reshape(2, 2, C, HW).transpose(0, 1, 3, 2)
    pc0_feats = pc0f[:2, :N, :]
    return dense_4d, pc0_feats


# speedup vs baseline: 4.1311x; 1.0494x over previous
import functools

import jax
import jax.numpy as jnp
from jax import lax
from jax.experimental import pallas as pl
from jax.experimental.pallas import tpu as pltpu
from jax.experimental.pallas import tpu_sc as plsc

N = 100000
NP = 100096
HW = 256 * 256
C = 32
L = 16
TV = 391
TP = TV * L
NV = N // L
ROWS = HW // 16

_mesh = plsc.VectorSubcoreMesh(core_axis_name="c", subcore_axis_name="s",
                               num_cores=2, num_subcores=16)
_sc_params = pltpu.CompilerParams(needs_layout_passes=False,
                                  use_tc_tiling_on_sc=False)


def _flat_from_xy(x, y):
    cx = ((x + 3.2) / 0.025).astype(jnp.int32)
    cy = ((y + 3.2) / 0.025).astype(jnp.int32)
    cx = jnp.clip(cx, 0, 255)
    cy = jnp.clip(cy, 0, 255)
    return cx * 256 + cy


@functools.partial(
    pl.kernel,
    mesh=_mesh,
    compiler_params=_sc_params,
    out_type=[
        jax.ShapeDtypeStruct((4, NP), jnp.int32),
        jax.ShapeDtypeStruct((4, 5, NP), jnp.float32),
        jax.ShapeDtypeStruct((4, 16, 32), jnp.int32),
    ],
    scratch_types=[
        pltpu.VMEM((HW + L,), jnp.float32),
        pltpu.VMEM((TP,), jnp.float32),
        pltpu.VMEM((TP,), jnp.float32),
        pltpu.VMEM((TP,), jnp.float32),
        pltpu.VMEM((TP,), jnp.int32),
        pltpu.VMEM((TP,), jnp.float32),
        pltpu.VMEM((ROWS,), jnp.float32),
        pltpu.VMEM((ROWS,), jnp.float32),
        pltpu.VMEM((ROWS,), jnp.float32),
        pltpu.VMEM((ROWS,), jnp.float32),
        pltpu.VMEM((32,), jnp.int32),
        pltpu.VMEM_SHARED((8, HW // 2), jnp.float32),
    ],
)
def _stats_kernel(pts, flat_out, f5_out, flags_out,
                  acc, xb, yb, zb, flb, ob, cnt, sx, sy, sz, flagbuf, S):
    cid = lax.axis_index("c")
    sid = lax.axis_index("s")
    base = pl.multiple_of(sid * TP, L)
    valid_bound = jnp.minimum(jnp.maximum(N - sid * TP, 0), TP)
    HH = HW // 2
    myhalf = sid // 8
    lrow = pl.multiple_of((sid % 8) * ROWS, L)

    def cloud_body(i, _):
        cloud = cid * 2 + i
        pltpu.sync_copy(pts.at[cloud, 0, pl.ds(base, TP)], xb)
        pltpu.sync_copy(pts.at[cloud, 1, pl.ds(base, TP)], yb)
        pltpu.sync_copy(pts.at[cloud, 2, pl.ds(base, TP)], zb)

        lane = jnp.arange(L, dtype=jnp.int32)

        def do_vreg(j):
            o = pl.multiple_of(j * L, L)
            fl = _flat_from_xy(xb[pl.ds(o, L)], yb[pl.ds(o, L)])
            m = (o + lane) < valid_bound
            fl = jnp.where(m, fl, HW)
            flb[pl.ds(o, L)] = fl
            _, lastm = plsc.scan_count(fl)
            return 16 - plsc.all_reduce_population_count(lastm)

        flagbuf[pl.ds(16, L)] = jnp.zeros((L,), jnp.int32)

        def win_body(g, _):
            acc16 = jnp.zeros((L,), jnp.int32)
            for u in range(16):
                acc16 = acc16 + do_vreg(g * 16 + u)
            plsc.store_scatter(flagbuf, [jnp.full((L,), 0, jnp.int32) + g],
                               acc16, mask=lane == 0)
            return 0
        lax.fori_loop(0, TV // 16, win_body, 0)
        acc16 = jnp.zeros((L,), jnp.int32)
        for u in range(TV % 16):
            acc16 = acc16 + do_vreg((TV // 16) * 16 + u)
        plsc.store_scatter(flagbuf, [jnp.full((L,), TV // 16, jnp.int32)],
                           acc16, mask=lane == 0)
        pltpu.sync_copy(flagbuf, flags_out.at[cloud, sid])

        for comp in range(4):
            dst = (cnt, sx, sy, sz)[comp]

            def z_body(j, _):
                for u in range(8):
                    o = pl.multiple_of(j * (8 * L) + u * L, L)
                    acc[pl.ds(o, L)] = jnp.zeros((L,), jnp.float32)
                return 0
            lax.fori_loop(0, HW // (8 * L), z_body, 0)

            def sc_body(j, _):
                o = pl.multiple_of(j * L, L)
                iv = flb[pl.ds(o, L)]
                if comp == 0:
                    vv = jnp.ones((L,), jnp.float32)
                elif comp == 1:
                    vv = xb[pl.ds(o, L)]
                elif comp == 2:
                    vv = yb[pl.ds(o, L)]
                else:
                    vv = zb[pl.ds(o, L)]
                m = iv < HW
                plsc.addupdate_scatter(acc, [iv], vv, mask=m)
                return 0
            lax.fori_loop(0, TV, sc_body, 0)

            for half in range(2):
                for rnd in range(2):
                    @pl.when((sid // 8) == rnd)
                    def _():
                        pltpu.sync_copy(acc.at[pl.ds(half * HH, HH)],
                                        S.at[sid % 8])
                    plsc.subcore_barrier()

                    @pl.when(myhalf == half)
                    def _():
                        for j in range(8):
                            pltpu.sync_copy(S.at[j, pl.ds(lrow, ROWS)], ob.at[pl.ds(0, ROWS)])

                            def mg_body(t, _):
                                for u in range(4):
                                    o = pl.multiple_of(t * (4 * L) + u * L, L)
                                    if rnd == 0 and j == 0:
                                        dst[pl.ds(o, L)] = ob[pl.ds(o, L)]
                                    else:
                                        dst[pl.ds(o, L)] = (dst[pl.ds(o, L)]
                                                            + ob[pl.ds(o, L)])
                                return 0
                            lax.fori_loop(0, ROWS // (4 * L), mg_body, 0)
                    plsc.subcore_barrier()

        def mean_body(t, _):
            o = pl.multiple_of(t * L, L)
            d = jnp.maximum(cnt[pl.ds(o, L)], 1.0)
            sx[pl.ds(o, L)] = sx[pl.ds(o, L)] / d
            sy[pl.ds(o, L)] = sy[pl.ds(o, L)] / d
            sz[pl.ds(o, L)] = sz[pl.ds(o, L)] / d
            return 0
        lax.fori_loop(0, ROWS // L, mean_body, 0)
        pltpu.sync_copy(sx, S.at[0 * 2 + myhalf, pl.ds(lrow, ROWS)])
        pltpu.sync_copy(sy, S.at[1 * 2 + myhalf, pl.ds(lrow, ROWS)])
        pltpu.sync_copy(sz, S.at[2 * 2 + myhalf, pl.ds(lrow, ROWS)])
        plsc.subcore_barrier()

        for comp in range(3):
            src = (xb, yb, zb)[comp]
            pltpu.sync_copy(S.at[comp * 2], acc.at[pl.ds(0, HH)])
            pltpu.sync_copy(S.at[comp * 2 + 1], acc.at[pl.ds(HH, HH)])

            def g_body(j, _):
                o = pl.multiple_of(j * L, L)
                iv = flb[pl.ds(o, L)]
                g = plsc.load_gather(acc, [iv])
                ob[pl.ds(o, L)] = src[pl.ds(o, L)] - g
                return 0
            lax.fori_loop(0, TV, g_body, 0)
            pltpu.sync_copy(ob, f5_out.at[cloud, comp, pl.ds(base, TP)])

        def fc_body(j, _):
            o = pl.multiple_of(j * L, L)
            fl = flb[pl.ds(o, L)]
            cxf = (fl >> 8).astype(jnp.float32)
            ob[pl.ds(o, L)] = xb[pl.ds(o, L)] - ((cxf + 0.5) * 0.025 + (-3.2))
            return 0
        lax.fori_loop(0, TV, fc_body, 0)
        pltpu.sync_copy(ob, f5_out.at[cloud, 3, pl.ds(base, TP)])

        def fcy_body(j, _):
            o = pl.multiple_of(j * L, L)
            fl = flb[pl.ds(o, L)]
            cyf = (fl & 255).astype(jnp.float32)
            ob[pl.ds(o, L)] = yb[pl.ds(o, L)] - ((cyf + 0.5) * 0.025 + (-3.2))
            return 0
        lax.fori_loop(0, TV, fcy_body, 0)
        pltpu.sync_copy(ob, f5_out.at[cloud, 4, pl.ds(base, TP)])

        pltpu.sync_copy(flb, flat_out.at[cloud, pl.ds(base, TP)])
        plsc.subcore_barrier()
        return 0

    lax.fori_loop(0, 2, cloud_body, 0)


BLK = 8192


def _mm_kernel(pts_ref, f5_ref, w_ref, b_ref, ptfT_ref, pc0_ref):
    x3 = pts_ref[0]
    f5 = f5_ref[0]
    x8 = jnp.concatenate([x3, f5], axis=0)
    w = w_ref[...]
    out = jnp.broadcast_to(b_ref[...].reshape(C, 1), (C, x8.shape[1]))
    for k in range(8):
        out = out + w[:, k:k + 1] * x8[k:k + 1, :]
    out = jnp.maximum(out, 0.0)
    ptfT_ref[0] = out
    pc0_ref[0] = out.T


def _matmul(pts_soa, f5, W, b):
    W8 = jnp.concatenate(
        [W[0:2], (W[2:3] + W[8:9]), W[3:8]], axis=0)
    W8T = W8.T
    nb = (NP + BLK - 1) // BLK
    return pl.pallas_call(
        _mm_kernel,
        grid=(4, nb),
        in_specs=[
            pl.BlockSpec((1, 3, BLK), lambda c, j: (c, 0, j)),
            pl.BlockSpec((1, 5, BLK), lambda c, j: (c, 0, j)),
            pl.BlockSpec((C, 8), lambda c, j: (0, 0)),
            pl.BlockSpec((1, C), lambda c, j: (0, 0)),
        ],
        out_specs=[
            pl.BlockSpec((1, C, BLK), lambda c, j: (c, 0, j)),
            pl.BlockSpec((1, BLK, C), lambda c, j: (c, j, 0)),
        ],
        out_shape=[
            jax.ShapeDtypeStruct((4, C, NP), jnp.float32),
            jax.ShapeDtypeStruct((4, NP, C), jnp.float32),
        ],
    )(pts_soa, f5, W8T, b.reshape(1, C))


SLAB = HW + L


@functools.partial(
    pl.kernel,
    mesh=_mesh,
    compiler_params=_sc_params,
    out_type=jax.ShapeDtypeStruct((4, C, HW), jnp.float32),
    scratch_types=[
        pltpu.VMEM((SLAB,), jnp.float32),
        pltpu.VMEM((TP,), jnp.int32),
        pltpu.VMEM((TP,), jnp.int32),
        pltpu.VMEM((TP,), jnp.float32),
        pltpu.VMEM((TP,), jnp.float32),
        pltpu.VMEM((32,), jnp.int32),
        pltpu.VMEM((32,), jnp.int32),
        pltpu.SemaphoreType.DMA,
        pltpu.SemaphoreType.DMA,
    ],
)
def _smax_kernel(flat, ptfT, flags, dense_out, slab,
                 flb0, flb1, vb0, vb1, fg0, fg1, sem0, sem1):
    cid = lax.axis_index("c")
    sid = lax.axis_index("s")
    ch = sid * 2 + cid
    bufs = ((flb0, vb0, fg0, sem0), (flb1, vb1, fg1, sem1))

    def issue(cloud, t, b):
        flb, vb, fgv, sem = bufs[b]
        cbase = pl.multiple_of(t * TP, L)
        pltpu.async_copy(flat.at[cloud, pl.ds(cbase, TP)], flb, sem)
        pltpu.async_copy(ptfT.at[cloud, ch, pl.ds(cbase, TP)], vb, sem)
        pltpu.async_copy(flags.at[cloud, t], fgv, sem)

    def drain(cloud, t, b):
        flb, vb, fgv, sem = bufs[b]
        cbase = pl.multiple_of(t * TP, L)
        pltpu.make_async_copy(flat.at[cloud, pl.ds(cbase, TP)], flb, sem).wait()
        pltpu.make_async_copy(ptfT.at[cloud, ch, pl.ds(cbase, TP)], vb, sem).wait()
        pltpu.make_async_copy(flags.at[cloud, t], fgv, sem).wait()

    def fix_vreg(flb, vb, o):
        iv = flb[pl.ds(o, L)]
        vv = vb[pl.ds(o, L)]
        cur = plsc.load_gather(slab, [iv])

        def cond(cur):
            return jnp.any(cur < vv)

        def body(cur):
            plsc.store_scatter(slab, [iv], jnp.maximum(cur, vv), mask=cur < vv)
            return plsc.load_gather(slab, [iv])

        lax.while_loop(cond, body, cur)

    def rmw_vreg(flb, vb, o):
        iv = flb[pl.ds(o, L)]
        vv = vb[pl.ds(o, L)]
        cur = plsc.load_gather(slab, [iv])
        plsc.store_scatter(slab, [iv], jnp.maximum(cur, vv))

    def process(b):
        flb, vb, fgv, _ = bufs[b]

        def v_body(g, _):
            for u in range(16):
                rmw_vreg(flb, vb, pl.multiple_of((g * 16 + u) * L, L))
            fg = plsc.load_gather(fgv, [jnp.zeros((L,), jnp.int32) + g])

            @pl.when(fg[0] > 0)
            def _():
                def fix_body(u, _):
                    fix_vreg(flb, vb, pl.multiple_of((g * 16 + u) * L, L))
                    return 0
                lax.fori_loop(0, 16, fix_body, 0)
            return 0
        lax.fori_loop(0, TV // 16, v_body, 0)
        for u in range(TV % 16):
            rmw_vreg(flb, vb, ((TV // 16) * 16 + u) * L)
        fg = plsc.load_gather(fgv, [jnp.full((L,), TV // 16, jnp.int32)])

        @pl.when(fg[0] > 0)
        def _():
            def fix_body(u, _):
                fix_vreg(flb, vb, pl.multiple_of(((TV // 16) * 16 + u) * L, L))
                return 0
            lax.fori_loop(0, TV % 16, fix_body, 0)

    def cloud_body(cloud, _):
        def z_body(j, _):
            for u in range(8):
                o = pl.multiple_of(j * (8 * L) + u * L, L)
                slab[pl.ds(o, L)] = jnp.zeros((L,), jnp.float32)
            return 0
        lax.fori_loop(0, SLAB // (8 * L), z_body, 0)

        issue(cloud, 0, 0)

        def pair_body(p, _):
            for b in range(2):
                t = p * 2 + b
                drain(cloud, t, b)

                @pl.when(t + 1 < 16)
                def _():
                    issue(cloud, t + 1, 1 - b)
                process(b)
            return 0
        lax.fori_loop(0, 8, pair_body, 0)

        pltpu.sync_copy(slab.at[pl.ds(0, HW)], dense_out.at[cloud, ch])
        return 0

    lax.fori_loop(0, 4, cloud_body, 0)


def kernel(pc0s, pc1s, W, b):
    pts_all = jnp.concatenate([pc0s, pc1s], axis=0)
    pts_soa = jnp.transpose(pts_all, (0, 2, 1))
    pts_soa = jnp.pad(pts_soa, ((0, 0), (0, 0), (0, NP - N)))
    flat, f5, flags = _stats_kernel(pts_soa)
    ptfT, pc0f = _matmul(pts_soa, f5, W, b)
    denseT = _smax_kernel(flat, ptfT, flags)
    dense_4d = denseT.reshape(2, 2, C, HW).transpose(0, 1, 3, 2)
    pc0_feats = pc0f[:2, :N, :]
    return dense_4d, pc0_feats
